# Initial kernel scaffold; baseline (speedup 1.0000x reference)
import jax
import jax.numpy as jnp
from jax.experimental import pallas as pl


def kernel(input_ids, embedding):
    raise NotImplementedError("write your pallas kernel here")


# speedup vs baseline: 1.7237x; 1.7237x over previous
import functools

import jax
import jax.numpy as jnp
from jax import lax
from jax.experimental import pallas as pl
from jax.experimental.pallas import tpu as pltpu
from jax.experimental.pallas import tpu_sc as plsc

NBUF = 4
CHUNK = 8


def _emb_lookup(n_total: int, d: int, num_workers: int):
    n_per_w = n_total // num_workers
    n_chunks = n_per_w // CHUNK
    n_groups = n_chunks // NBUF
    mesh = plsc.VectorSubcoreMesh(core_axis_name="c", subcore_axis_name="s")

    @functools.partial(
        pl.kernel,
        mesh=mesh,
        out_type=jax.ShapeDtypeStruct((n_total, d), jnp.float32),
        scratch_types=(
            [pltpu.VMEM((n_per_w,), jnp.int32)]
            + [pltpu.VMEM((CHUNK, d), jnp.float32) for _ in range(NBUF)]
            + [pltpu.SemaphoreType.DMA for _ in range(2 * NBUF)]
        ),
    )
    def emb_kernel(ids_hbm, table_hbm, out_hbm, idx_v, *rest):
        bufs = rest[:NBUF]
        gsems = rest[NBUF:2 * NBUF]
        osems = rest[2 * NBUF:]

        wid = lax.axis_index("s") * 2 + lax.axis_index("c")
        base = wid * n_per_w
        pltpu.sync_copy(ids_hbm.at[pl.ds(base, n_per_w)], idx_v)

        def gather_desc(c, b):
            off = pl.multiple_of(c * CHUNK, 8)
            return pltpu.make_async_copy(
                table_hbm.at[idx_v.at[pl.ds(off, CHUNK)]], bufs[b], gsems[b])

        def out_desc(c, b):
            off = pl.multiple_of(base + c * CHUNK, 8)
            return pltpu.make_async_copy(
                bufs[b], out_hbm.at[pl.ds(off, CHUNK)], osems[b])

        for b in range(NBUF):
            gather_desc(b, b).start()

        def body(i, _):
            g = i * NBUF
            for b in range(NBUF):
                gather_desc(g + b, b).wait()
                out_desc(g + b, b).start()
            for b in range(NBUF):
                out_desc(g + b, b).wait()
                gather_desc(g + NBUF + b, b).start()
            return 0

        lax.fori_loop(0, n_groups - 1, body, 0)

        g = (n_groups - 1) * NBUF
        for b in range(NBUF):
            gather_desc(g + b, b).wait()
            out_desc(g + b, b).start()
        for b in range(NBUF):
            out_desc(g + b, b).wait()

    return emb_kernel


def kernel(input_ids, embedding):
    b, s = input_ids.shape
    v, d = embedding.shape
    n = b * s
    ids = input_ids.reshape(n).astype(jnp.int32)
    out = _emb_lookup(n, d, 32)(ids, embedding)
    return out.reshape(b, s, d)
